# Initial kernel scaffold; baseline (speedup 1.0000x reference)
import jax
import jax.numpy as jnp
from jax.experimental import pallas as pl


def kernel(embeddings, modality_table, token_type_id):
    raise NotImplementedError("write your pallas kernel here")


# speedup vs baseline: 3.3329x; 3.3329x over previous
import jax
import jax.numpy as jnp
from jax.experimental import pallas as pl
from jax.experimental.pallas import tpu as pltpu

BLOCK_ROWS = 512


def _body(tid_ref, table_ref, out_ref):
    tid = tid_ref[0]
    r0 = table_ref[0, :]
    r1 = table_ref[1, :]
    r2 = table_ref[2, :]
    row = jnp.where(tid == 0, r0, jnp.where(tid == 1, r1, r2))
    out_ref[...] = jnp.broadcast_to(row[None, :], out_ref.shape)


def kernel(embeddings, modality_table, token_type_id):
    seq_len = embeddings.shape[1]
    d_model = modality_table.shape[1]
    tid = jnp.asarray(token_type_id, dtype=jnp.int32).reshape((1,))
    grid = (seq_len // BLOCK_ROWS,)
    out = pl.pallas_call(
        _body,
        grid_spec=pltpu.PrefetchScalarGridSpec(
            num_scalar_prefetch=1,
            grid=grid,
            in_specs=[
                pl.BlockSpec(modality_table.shape, lambda i, tid: (0, 0)),
            ],
            out_specs=pl.BlockSpec((BLOCK_ROWS, d_model), lambda i, tid: (i, 0)),
        ),
        out_shape=jax.ShapeDtypeStruct((seq_len, d_model), jnp.float32),
    )(tid, modality_table)
    return out
